# baseline (speedup 1.0000x reference)
import functools

import jax
import jax.numpy as jnp
from jax import lax
from jax.experimental import pallas as pl
from jax.experimental.pallas import tpu as pltpu
from jax.experimental.pallas import tpu_sc as plsc

BATCH = 4096
N_FIELDS = 26
EMBED_DIM = 64
B = BATCH * N_FIELDS
NW = 32
BPW = B // NW
CH = 128
NCH = BPW // CH
NBUF = 4


def kernel(indices, table):
    idx = indices.reshape(B).astype(jnp.int32)
    qidx = (idx >> 1).reshape(NW, NCH, CH)
    table_pairs = table.reshape(-1, 2 * EMBED_DIM)

    mesh = plsc.VectorSubcoreMesh(core_axis_name="c", subcore_axis_name="s")

    @functools.partial(
        pl.kernel,
        out_type=jax.ShapeDtypeStruct((B, 2 * EMBED_DIM), jnp.float32),
        mesh=mesh,
        scratch_types=[
            pltpu.VMEM((NCH, CH), jnp.int32),
            pltpu.VMEM((NBUF, CH, 2 * EMBED_DIM), jnp.float32),
            pltpu.SemaphoreType.DMA,
            pltpu.SemaphoreType.DMA((NBUF,)),
            pltpu.SemaphoreType.DMA((NBUF,)),
        ],
    )
    def gather_kernel(table_hbm, idx_hbm, out_hbm, idx_v, rows_v, isem, gsem, wsem):
        wid = lax.axis_index("s") * 2 + lax.axis_index("c")
        base = wid * BPW
        cp = pltpu.make_async_copy(idx_hbm.at[wid], idx_v, isem)
        cp.start()
        cp.wait()

        def gather_cp(c, b):
            return pltpu.make_async_copy(
                table_hbm.at[idx_v.at[c]], rows_v.at[b], gsem.at[b]
            )

        def write_cp(c, b):
            return pltpu.make_async_copy(
                rows_v.at[b], out_hbm.at[pl.ds(base + c * CH, CH)], wsem.at[b]
            )

        for b in range(NBUF):
            gather_cp(b, b).start()

        for c in range(NCH):
            b = c % NBUF
            gather_cp(c, b).wait()
            write_cp(c, b).start()
            n = c + NBUF
            if n < NCH:
                write_cp(c, b).wait()
                gather_cp(n, b).start()

        for c in range(NCH - NBUF, NCH):
            b = c % NBUF
            write_cp(c, b).wait()

    pairs = gather_kernel(table_pairs, qidx)
    hi = (idx & 1).astype(bool)[:, None]
    out = jnp.where(hi, pairs[:, EMBED_DIM:], pairs[:, :EMBED_DIM])
    return out.reshape(BATCH, N_FIELDS, EMBED_DIM)


# speedup vs baseline: 1.7220x; 1.7220x over previous
import functools

import jax
import jax.numpy as jnp
from jax import lax
from jax.experimental import pallas as pl
from jax.experimental.pallas import tpu as pltpu
from jax.experimental.pallas import tpu_sc as plsc

BATCH = 4096
N_FIELDS = 26
EMBED_DIM = 64
B = BATCH * N_FIELDS
NW = 32
BPW = B // NW
CH = 128
NCH = BPW // CH
NBUF = 4


NUM_EMB = 1000000
BQ = 4096
NQBLK = -(-NUM_EMB // (2 * BQ))
NPAIR = NQBLK * BQ


def _pack_body(lo_ref, hi_ref, out_ref):
    out_ref[:, :EMBED_DIM] = jnp.swapaxes(lo_ref[...], 0, 1)
    out_ref[:, EMBED_DIM:] = jnp.swapaxes(hi_ref[...], 0, 1)


def _pack_pairs(table):
    table_t = table.T
    return pl.pallas_call(
        _pack_body,
        grid=(NQBLK,),
        in_specs=[
            pl.BlockSpec((EMBED_DIM, BQ), lambda i: (0, 2 * i)),
            pl.BlockSpec(
                (EMBED_DIM, BQ),
                lambda i: (0, jnp.minimum(2 * i + 1, NUM_EMB // BQ - 1)),
            ),
        ],
        out_specs=pl.BlockSpec((BQ, 2 * EMBED_DIM), lambda i: (i, 0)),
        out_shape=jax.ShapeDtypeStruct((NPAIR, 2 * EMBED_DIM), jnp.float32),
        compiler_params=pltpu.CompilerParams(dimension_semantics=("parallel",)),
    )(table_t, table_t)


def kernel(indices, table):
    idx = indices.reshape(B).astype(jnp.int32)
    blk, rem = idx // (2 * BQ), idx % (2 * BQ)
    hi_half, qq = rem // BQ, rem % BQ
    qidx = (blk * BQ + qq).reshape(NW, NCH, CH)
    table_pairs = _pack_pairs(table)

    mesh = plsc.VectorSubcoreMesh(core_axis_name="c", subcore_axis_name="s")

    @functools.partial(
        pl.kernel,
        out_type=jax.ShapeDtypeStruct((B, 2 * EMBED_DIM), jnp.float32),
        mesh=mesh,
        scratch_types=[
            pltpu.VMEM((NCH, CH), jnp.int32),
            pltpu.VMEM((NBUF, CH, 2 * EMBED_DIM), jnp.float32),
            pltpu.SemaphoreType.DMA,
            pltpu.SemaphoreType.DMA((NBUF,)),
            pltpu.SemaphoreType.DMA((NBUF,)),
        ],
    )
    def gather_kernel(table_hbm, idx_hbm, out_hbm, idx_v, rows_v, isem, gsem, wsem):
        wid = lax.axis_index("s") * 2 + lax.axis_index("c")
        base = wid * BPW
        cp = pltpu.make_async_copy(idx_hbm.at[wid], idx_v, isem)
        cp.start()
        cp.wait()

        def gather_cp(c, b):
            return pltpu.make_async_copy(
                table_hbm.at[idx_v.at[c]], rows_v.at[b], gsem.at[b]
            )

        def write_cp(c, b):
            return pltpu.make_async_copy(
                rows_v.at[b], out_hbm.at[pl.ds(base + c * CH, CH)], wsem.at[b]
            )

        for b in range(NBUF):
            gather_cp(b, b).start()

        for c in range(NCH):
            b = c % NBUF
            gather_cp(c, b).wait()
            write_cp(c, b).start()
            n = c + NBUF
            if n < NCH:
                write_cp(c, b).wait()
                gather_cp(n, b).start()

        for c in range(NCH - NBUF, NCH):
            b = c % NBUF
            write_cp(c, b).wait()

    pairs = gather_kernel(table_pairs, qidx)
    hi = hi_half.astype(bool)[:, None]
    out = jnp.where(hi, pairs[:, EMBED_DIM:], pairs[:, :EMBED_DIM])
    return out.reshape(BATCH, N_FIELDS, EMBED_DIM)


# speedup vs baseline: 2.1712x; 1.2609x over previous
import functools

import jax
import jax.numpy as jnp
from jax import lax
from jax.experimental import pallas as pl
from jax.experimental.pallas import tpu as pltpu
from jax.experimental.pallas import tpu_sc as plsc

BATCH = 4096
N_FIELDS = 26
EMBED_DIM = 64
B = BATCH * N_FIELDS
NW = 32
BPW = B // NW
CH = 128
NCH = BPW // CH
NBUF = 4


NUM_EMB = 1000000
BQ = 4096
NQBLK = -(-NUM_EMB // (2 * BQ))
NPAIR = NQBLK * BQ


def _pack_body(lo_ref, hi_ref, out_ref):
    out_ref[:, :EMBED_DIM] = jnp.swapaxes(lo_ref[...], 0, 1)
    out_ref[:, EMBED_DIM:] = jnp.swapaxes(hi_ref[...], 0, 1)


def _pack_pairs(table):
    table_t = table.T
    return pl.pallas_call(
        _pack_body,
        grid=(NQBLK,),
        in_specs=[
            pl.BlockSpec((EMBED_DIM, BQ), lambda i: (0, 2 * i)),
            pl.BlockSpec(
                (EMBED_DIM, BQ),
                lambda i: (0, jnp.minimum(2 * i + 1, NUM_EMB // BQ - 1)),
            ),
        ],
        out_specs=pl.BlockSpec((BQ, 2 * EMBED_DIM), lambda i: (i, 0)),
        out_shape=jax.ShapeDtypeStruct((NPAIR, 2 * EMBED_DIM), jnp.float32),
        compiler_params=pltpu.CompilerParams(dimension_semantics=("parallel",)),
    )(table_t, table_t)


def _select_body(pairs_ref, p_ref, out_ref):
    x = pairs_ref[...]
    pcol = p_ref[...][0, 0][:, None]
    sel = jnp.where(pcol > 0, x[:, EMBED_DIM:], x[:, :EMBED_DIM])
    out_ref[...] = jnp.swapaxes(sel, 0, 1)[None]


def _select_pack_out(pairs, hi_half):
    return pl.pallas_call(
        _select_body,
        grid=(N_FIELDS,),
        in_specs=[
            pl.BlockSpec((BATCH, 2 * EMBED_DIM), lambda i: (i, 0)),
            pl.BlockSpec((1, 1, BATCH), lambda i: (i, 0, 0)),
        ],
        out_specs=pl.BlockSpec((1, EMBED_DIM, BATCH), lambda i: (i, 0, 0)),
        out_shape=jax.ShapeDtypeStruct((N_FIELDS, EMBED_DIM, BATCH), jnp.float32),
        compiler_params=pltpu.CompilerParams(dimension_semantics=("parallel",)),
    )(pairs, hi_half.reshape(N_FIELDS, 1, BATCH))


def kernel(indices, table):
    idx = indices.T.reshape(B).astype(jnp.int32)
    blk, rem = idx // (2 * BQ), idx % (2 * BQ)
    hi_half, qq = rem // BQ, rem % BQ
    qidx = (blk * BQ + qq).reshape(NW, NCH, CH)
    table_pairs = _pack_pairs(table)

    mesh = plsc.VectorSubcoreMesh(core_axis_name="c", subcore_axis_name="s")

    @functools.partial(
        pl.kernel,
        out_type=jax.ShapeDtypeStruct((B, 2 * EMBED_DIM), jnp.float32),
        mesh=mesh,
        scratch_types=[
            pltpu.VMEM((NCH, CH), jnp.int32),
            pltpu.VMEM((NBUF, CH, 2 * EMBED_DIM), jnp.float32),
            pltpu.SemaphoreType.DMA,
            pltpu.SemaphoreType.DMA((NBUF,)),
            pltpu.SemaphoreType.DMA((NBUF,)),
        ],
    )
    def gather_kernel(table_hbm, idx_hbm, out_hbm, idx_v, rows_v, isem, gsem, wsem):
        wid = lax.axis_index("s") * 2 + lax.axis_index("c")
        base = wid * BPW
        cp = pltpu.make_async_copy(idx_hbm.at[wid], idx_v, isem)
        cp.start()
        cp.wait()

        def gather_cp(c, b):
            return pltpu.make_async_copy(
                table_hbm.at[idx_v.at[c]], rows_v.at[b], gsem.at[b]
            )

        def write_cp(c, b):
            return pltpu.make_async_copy(
                rows_v.at[b], out_hbm.at[pl.ds(base + c * CH, CH)], wsem.at[b]
            )

        for b in range(NBUF):
            gather_cp(b, b).start()

        for c in range(NCH):
            b = c % NBUF
            gather_cp(c, b).wait()
            write_cp(c, b).start()
            n = c + NBUF
            if n < NCH:
                write_cp(c, b).wait()
                gather_cp(n, b).start()

        for c in range(NCH - NBUF, NCH):
            b = c % NBUF
            write_cp(c, b).wait()

    pairs = gather_kernel(table_pairs, qidx)
    out_t = _select_pack_out(pairs, hi_half)
    return out_t.transpose(2, 0, 1)


# speedup vs baseline: 2.3700x; 1.0915x over previous
import functools

import jax
import jax.numpy as jnp
from jax import lax
from jax.experimental import pallas as pl
from jax.experimental.pallas import tpu as pltpu
from jax.experimental.pallas import tpu_sc as plsc

BATCH = 4096
N_FIELDS = 26
EMBED_DIM = 64
B = BATCH * N_FIELDS
NW = 32
BPW = B // NW
CH = 128
NCH = BPW // CH
NBUF = 6


NUM_EMB = 1000000
BQ = 8192
NQBLK = -(-NUM_EMB // (2 * BQ))
NPAIR = NQBLK * BQ


def _pack_body(lo_ref, hi_ref, out_ref):
    out_ref[:, :EMBED_DIM] = jnp.swapaxes(lo_ref[...], 0, 1)
    out_ref[:, EMBED_DIM:] = jnp.swapaxes(hi_ref[...], 0, 1)


def _pack_pairs(table):
    table_t = table.T
    return pl.pallas_call(
        _pack_body,
        grid=(NQBLK,),
        in_specs=[
            pl.BlockSpec((EMBED_DIM, BQ), lambda i: (0, 2 * i)),
            pl.BlockSpec(
                (EMBED_DIM, BQ),
                lambda i: (0, jnp.minimum(2 * i + 1, NUM_EMB // BQ - 1)),
            ),
        ],
        out_specs=pl.BlockSpec((BQ, 2 * EMBED_DIM), lambda i: (i, 0)),
        out_shape=jax.ShapeDtypeStruct((NPAIR, 2 * EMBED_DIM), jnp.float32),
        compiler_params=pltpu.CompilerParams(dimension_semantics=("parallel",)),
    )(table_t, table_t)


def _select_body(pairs_ref, p_ref, out_ref):
    x = pairs_ref[...]
    pcol = p_ref[...][0, 0][:, None]
    sel = jnp.where(pcol > 0, x[:, EMBED_DIM:], x[:, :EMBED_DIM])
    out_ref[...] = jnp.swapaxes(sel, 0, 1)[None]


def _select_pack_out(pairs, hi_half):
    return pl.pallas_call(
        _select_body,
        grid=(N_FIELDS,),
        in_specs=[
            pl.BlockSpec((BATCH, 2 * EMBED_DIM), lambda i: (i, 0)),
            pl.BlockSpec((1, 1, BATCH), lambda i: (i, 0, 0)),
        ],
        out_specs=pl.BlockSpec((1, EMBED_DIM, BATCH), lambda i: (i, 0, 0)),
        out_shape=jax.ShapeDtypeStruct((N_FIELDS, EMBED_DIM, BATCH), jnp.float32),
        compiler_params=pltpu.CompilerParams(dimension_semantics=("parallel",)),
    )(pairs, hi_half.reshape(N_FIELDS, 1, BATCH))


def kernel(indices, table):
    idx = indices.T.reshape(B).astype(jnp.int32)
    blk, rem = idx // (2 * BQ), idx % (2 * BQ)
    hi_half, qq = rem // BQ, rem % BQ
    qidx = (blk * BQ + qq).reshape(NW, NCH, CH)
    table_pairs = _pack_pairs(table)

    mesh = plsc.VectorSubcoreMesh(core_axis_name="c", subcore_axis_name="s")

    @functools.partial(
        pl.kernel,
        out_type=jax.ShapeDtypeStruct((B, 2 * EMBED_DIM), jnp.float32),
        mesh=mesh,
        scratch_types=[
            pltpu.VMEM((NCH, CH), jnp.int32),
            pltpu.VMEM((NBUF, CH, 2 * EMBED_DIM), jnp.float32),
            pltpu.SemaphoreType.DMA,
            pltpu.SemaphoreType.DMA((NBUF,)),
            pltpu.SemaphoreType.DMA((NBUF,)),
        ],
    )
    def gather_kernel(table_hbm, idx_hbm, out_hbm, idx_v, rows_v, isem, gsem, wsem):
        wid = lax.axis_index("s") * 2 + lax.axis_index("c")
        base = wid * BPW
        cp = pltpu.make_async_copy(idx_hbm.at[wid], idx_v, isem)
        cp.start()
        cp.wait()

        def gather_cp(c, b):
            return pltpu.make_async_copy(
                table_hbm.at[idx_v.at[c]], rows_v.at[b], gsem.at[b]
            )

        def write_cp(c, b):
            return pltpu.make_async_copy(
                rows_v.at[b], out_hbm.at[pl.ds(base + c * CH, CH)], wsem.at[b]
            )

        for b in range(NBUF):
            gather_cp(b, b).start()

        for c in range(NCH):
            b = c % NBUF
            gather_cp(c, b).wait()
            write_cp(c, b).start()
            n = c + NBUF
            if n < NCH:
                write_cp(c, b).wait()
                gather_cp(n, b).start()

        for c in range(NCH - NBUF, NCH):
            b = c % NBUF
            write_cp(c, b).wait()

    pairs = gather_kernel(table_pairs, qidx)
    out_t = _select_pack_out(pairs, hi_half)
    return out_t.transpose(2, 0, 1)
